# Initial kernel scaffold; baseline (speedup 1.0000x reference)
import jax
import jax.numpy as jnp
from jax.experimental import pallas as pl


def kernel(points, W1, g1, b1, W2, g2, b2, Wc1, gc1, bc1, Wc2, bc2):
    raise NotImplementedError("write your pallas kernel here")


# speedup vs baseline: 6.3545x; 6.3545x over previous
import functools

import jax
import jax.numpy as jnp
from jax import lax
from jax.experimental import pallas as pl
from jax.experimental.pallas import tpu as pltpu
from jax.experimental.pallas import tpu_sc as plsc

_B, _N, _K = 8, 2048, 16
_BN = _B * _N
_ROWS = 256
_NBLK = _N // _ROWS
_NEG = -3.0e38
_BIG = 3.0e38
_F32 = jnp.float32

_NC, _NS, _L = 2, 16, 16
_NW = _NC * _NS
_PW = _BN // _NW
_P = 128
_CH = _PW // _P
_KH = 2
_KG = _K // _KH


def _knn_body(rows_ref, cols_ref, idx_ref, geo_ref, *, want_geom):
    b = pl.program_id(0)
    pr = rows_ref[0]
    ca = cols_ref[0]
    r0, r1, r2 = pr[:, 0:1], pr[:, 1:2], pr[:, 2:3]
    c0, c1, c2 = ca[0:1, :], ca[1:2, :], ca[2:3, :]
    inner = jnp.dot(pr, ca, preferred_element_type=_F32,
                    precision=lax.Precision.DEFAULT)
    xxr = r0 * r0 + r1 * r1 + r2 * r2
    xxc = c0 * c0 + c1 * c1 + c2 * c2
    D = 2.0 * inner - xxr - xxc
    col = lax.broadcasted_iota(jnp.int32, (_ROWS, _N), 1)
    base = b * _N
    ds = []
    for _t in range(_K):
        m = jnp.max(D, axis=1, keepdims=True)
        j = jnp.min(jnp.where(D == m, col, _N), axis=1, keepdims=True)
        sel = col == j
        idx_ref[0, :, _t:_t + 1] = j + base
        if want_geom:
            s0 = jnp.min(jnp.where(sel, c0, _BIG), axis=1, keepdims=True)
            s1 = jnp.min(jnp.where(sel, c1, _BIG), axis=1, keepdims=True)
            s2 = jnp.min(jnp.where(sel, c2, _BIG), axis=1, keepdims=True)
            d0, d1, d2 = s0 - r0, s1 - r1, s2 - r2
            dsq = d0 * d0 + d1 * d1 + d2 * d2
            ds.append(jnp.sqrt(dsq + 1e-8))
        D = jnp.where(sel, _NEG, D)
    if want_geom:
        tot = ds[0]
        for t in range(1, _K):
            tot = tot + ds[t]
        mean = tot / float(_K)
        vtot = (ds[0] - mean) * (ds[0] - mean)
        for t in range(1, _K):
            dv = ds[t] - mean
            vtot = vtot + dv * dv
        std = jnp.sqrt(vtot / float(_K))
        geo_ref[0, :, 0:1] = mean
        geo_ref[0, :, 1:2] = std


def _knn_call(rows3, cols3, want_geom):
    grid = (_B, _NBLK)
    in_specs = [
        pl.BlockSpec((1, _ROWS, 3), lambda b, r: (b, r, 0)),
        pl.BlockSpec((1, 3, _N), lambda b, r: (b, 0, 0)),
    ]
    idx_shape = jax.ShapeDtypeStruct((_B * _NBLK, _ROWS, _K), jnp.int32)
    idx_spec = pl.BlockSpec((1, _ROWS, _K), lambda b, r: (b * _NBLK + r, 0, 0))
    geo_shape = jax.ShapeDtypeStruct((_B * _NBLK, _ROWS, 2), _F32)
    geo_spec = pl.BlockSpec((1, _ROWS, 2), lambda b, r: (b * _NBLK + r, 0, 0))
    if want_geom:
        body = functools.partial(_knn_body, want_geom=True)
        idx, geo = pl.pallas_call(
            body, grid=grid, in_specs=in_specs,
            out_specs=[idx_spec, geo_spec],
            out_shape=[idx_shape, geo_shape])(rows3, cols3)
    else:
        def body(rows_ref, cols_ref, idx_ref):
            _knn_body(rows_ref, cols_ref, idx_ref, None, want_geom=False)
        idx = pl.pallas_call(
            body, grid=grid, in_specs=in_specs,
            out_specs=idx_spec, out_shape=idx_shape)(rows3, cols3)
        geo = None
    idx_t = idx.reshape(_B, _N, _K).transpose(2, 0, 1).reshape(_K, _BN)
    if not want_geom:
        return idx_t
    geo = geo.reshape(_B, _N, 2)
    return idx_t, geo[..., 0], geo[..., 1]


def _sc_gather_body(idx_hbm, table_hbm, nb_o, idx_v, buf_v, sem):
    wid = lax.axis_index("s") * _NC + lax.axis_index("c")
    base = wid * _PW
    for chunk in range(_CH):
        pbase = base + chunk * _P
        pltpu.sync_copy(idx_hbm.at[:, pl.ds(pbase, _P)], idx_v)
        for grp in range(_KG):
            copies = [
                pltpu.async_copy(table_hbm.at[idx_v.at[grp * _KH + j]],
                                 buf_v.at[j], sem)
                for j in range(_KH)
            ]
            for cp in copies:
                cp.wait()
            for j in range(_KH):
                pltpu.sync_copy(
                    buf_v.at[j], nb_o.at[grp * _KH + j, pl.ds(pbase, _P)])


@functools.lru_cache(maxsize=1)
def _sc_gather_kernel():
    return pl.kernel(
        _sc_gather_body,
        out_type=jax.ShapeDtypeStruct((_K, _BN, 128), _F32),
        mesh=plsc.VectorSubcoreMesh(core_axis_name="c", subcore_axis_name="s"),
        scratch_types=[
            pltpu.VMEM((_K, _P), jnp.int32),
            pltpu.VMEM((_KH, _P, 128), _F32),
            pltpu.SemaphoreType.DMA,
        ],
    )


def _edge_gather(idx_t, table128):
    return _sc_gather_kernel()(idx_t, table128)


def _edge_body(nb_ref, c_ref, wt_ref, mx_ref, mn_ref, s_ref, *, ch):
    @pl.when(pl.program_id(0) == 0)
    def _():
        s_ref[...] = jnp.zeros_like(s_ref)

    nb = nb_ref[...][:, :, :ch]
    c = c_ref[...]
    cb = jnp.broadcast_to(c[None], (_K, _ROWS, ch))
    feats = jnp.concatenate([nb - cb, cb], axis=2)
    y = jnp.dot(feats.reshape(_K * _ROWS, 2 * ch), wt_ref[...],
                preferred_element_type=_F32, precision=lax.Precision.DEFAULT)
    yk = y.reshape(_K, _ROWS, 64)
    mx_ref[...] = jnp.max(yk, axis=0)
    mn_ref[...] = jnp.min(yk, axis=0)
    s_ref[0:1, :] += jnp.sum(y, axis=0, keepdims=True)
    s_ref[1:2, :] += jnp.sum(y * y, axis=0, keepdims=True)


def _edge_call(nb, x, wt):
    ch = x.shape[1]
    grid = (_BN // _ROWS,)
    body = functools.partial(_edge_body, ch=ch)
    spec64 = pl.BlockSpec((_ROWS, 64), lambda i: (i, 0))
    return pl.pallas_call(
        body, grid=grid,
        in_specs=[pl.BlockSpec((_K, _ROWS, 128), lambda i: (0, i, 0)),
                  pl.BlockSpec((_ROWS, ch), lambda i: (i, 0)),
                  pl.BlockSpec((2 * ch, 64), lambda i: (0, 0))],
        out_specs=[spec64, spec64, pl.BlockSpec((8, 64), lambda i: (0, 0))],
        out_shape=[jax.ShapeDtypeStruct((_BN, 64), _F32),
                   jax.ShapeDtypeStruct((_BN, 64), _F32),
                   jax.ShapeDtypeStruct((8, 64), _F32)])(nb, x, wt)


def _apply_body(mx_ref, mn_ref, sums_ref, g_ref, b_ref, x_ref, *, m):
    sy = sums_ref[0:1, :]
    sy2 = sums_ref[1:2, :]
    mean = sy / m
    var = sy2 / m - mean * mean
    a = g_ref[...] / jnp.sqrt(var + 1e-5)
    c = b_ref[...] - a * mean
    h = a * jnp.where(a >= 0.0, mx_ref[...], mn_ref[...]) + c
    x_ref[...] = jnp.where(h >= 0.0, h, 0.2 * h)


def _apply_call(mx, mn, sums, g, b):
    r = 2048
    grid = (_BN // r,)
    spec = pl.BlockSpec((r, 64), lambda i: (i, 0))
    body = functools.partial(_apply_body, m=float(_BN * _K))
    return pl.pallas_call(
        body, grid=grid,
        in_specs=[spec, spec,
                  pl.BlockSpec((8, 64), lambda i: (0, 0)),
                  pl.BlockSpec((1, 64), lambda i: (0, 0)),
                  pl.BlockSpec((1, 64), lambda i: (0, 0))],
        out_specs=spec,
        out_shape=jax.ShapeDtypeStruct((_BN, 64), _F32))(
            mx, mn, sums, g.reshape(1, 64), b.reshape(1, 64))


def _head1_body(x1_ref, x2_ref, wa_ref, wb_ref, h_ref, s_ref):
    @pl.when(pl.program_id(0) == 0)
    def _():
        s_ref[...] = jnp.zeros_like(s_ref)

    h = (jnp.dot(x1_ref[...], wa_ref[...], preferred_element_type=_F32,
                 precision=lax.Precision.DEFAULT) +
         jnp.dot(x2_ref[...], wb_ref[...], preferred_element_type=_F32,
                 precision=lax.Precision.DEFAULT))
    h_ref[...] = h
    s_ref[0:1, :] += jnp.sum(h, axis=0, keepdims=True)
    s_ref[1:2, :] += jnp.sum(h * h, axis=0, keepdims=True)


def _head1_call(x1, x2, wa, wb):
    r = 2048
    grid = (_BN // r,)
    spec = pl.BlockSpec((r, 64), lambda i: (i, 0))
    return pl.pallas_call(
        _head1_body, grid=grid,
        in_specs=[spec, spec,
                  pl.BlockSpec((64, 256), lambda i: (0, 0)),
                  pl.BlockSpec((64, 256), lambda i: (0, 0))],
        out_specs=[pl.BlockSpec((r, 256), lambda i: (i, 0)),
                   pl.BlockSpec((8, 256), lambda i: (0, 0))],
        out_shape=[jax.ShapeDtypeStruct((_BN, 256), _F32),
                   jax.ShapeDtypeStruct((8, 256), _F32)])(x1, x2, wa, wb)


def _head2_body(h_ref, s_ref, g_ref, b_ref, w_ref, bc_ref, o_ref, *, m):
    sy = s_ref[0:1, :]
    sy2 = s_ref[1:2, :]
    mean = sy / m
    var = sy2 / m - mean * mean
    a = g_ref[...] / jnp.sqrt(var + 1e-5)
    c = b_ref[...] - a * mean
    h = a * h_ref[...] + c
    h = jnp.where(h >= 0.0, h, 0.2 * h)
    o_ref[...] = jnp.dot(h, w_ref[...], preferred_element_type=_F32,
                         precision=lax.Precision.DEFAULT) + bc_ref[...]


def _head2_call(h3, sums, g, b, wt, bc):
    r = 2048
    grid = (_BN // r,)
    body = functools.partial(_head2_body, m=float(_BN))
    return pl.pallas_call(
        body, grid=grid,
        in_specs=[pl.BlockSpec((r, 256), lambda i: (i, 0)),
                  pl.BlockSpec((8, 256), lambda i: (0, 0)),
                  pl.BlockSpec((1, 256), lambda i: (0, 0)),
                  pl.BlockSpec((1, 256), lambda i: (0, 0)),
                  pl.BlockSpec((256, 2), lambda i: (0, 0)),
                  pl.BlockSpec((1, 2), lambda i: (0, 0))],
        out_specs=pl.BlockSpec((r, 2), lambda i: (i, 0)),
        out_shape=jax.ShapeDtypeStruct((_BN, 2), _F32))(
            h3, sums, g.reshape(1, 256), b.reshape(1, 256), wt,
            bc.reshape(1, 2))


def kernel(points, W1, g1, b1, W2, g2, b2, Wc1, gc1, bc1, Wc2, bc2):
    pts_t = points.transpose(0, 2, 1)
    idx1_t, meand, stdd = _knn_call(pts_t, points, True)
    x_t = jnp.concatenate(
        [pts_t, meand[..., None], stdd[..., None]], axis=2).reshape(_BN, 5)

    nb1 = _edge_gather(idx1_t, jnp.pad(x_t, ((0, 0), (0, 123))))
    mx1, mn1, sums1 = _edge_call(nb1, x_t, W1.T)
    x1 = _apply_call(mx1, mn1, sums1, g1, b1)

    x1r = x1.reshape(_B, _N, 64)
    rows3 = x1r[:, :, :3]
    idx2_t = _knn_call(rows3, rows3.transpose(0, 2, 1), False)

    nb2 = _edge_gather(idx2_t, jnp.pad(x1, ((0, 0), (0, 64))))
    mx2, mn2, sums2 = _edge_call(nb2, x1, W2.T)
    x2 = _apply_call(mx2, mn2, sums2, g2, b2)

    h3, hsums = _head1_call(x1, x2, Wc1[:, :64].T, Wc1[:, 64:].T)
    logits_t = _head2_call(h3, hsums, gc1, bc1, Wc2.T, bc2)
    return logits_t.reshape(_B, _N, 2).transpose(0, 2, 1)
